# baseline (speedup 1.0000x reference)
import functools

import jax
import jax.numpy as jnp
from jax import lax
from jax.experimental import pallas as pl
from jax.experimental.pallas import tpu as pltpu
from jax.experimental.pallas import tpu_sc as plsc

B = 1024
L = 50
D = 128
NLOOK = 5
NC, NS = 2, 16
NW = NC * NS
CHUNK_ROWS = 2
IDX_MINOR = 2 * L
DCH = D // 16


NBUF = 4


def _make_sc_body(bpw, nchunk):
    def body(c0, c1, c2, c3, c4, diag_t, proc_t, med_t, out_hbm,
             idx_v, rows0_v, rows1_v, rows2_v, rows3_v, acc_v,
             sem0, sem1, sem2, sem3, sem_idx, sem_out):
        wid = lax.axis_index("s") * NC + lax.axis_index("c")
        base = wid * bpw
        codes = (c0, c1, c2, c3, c4)
        tables = (diag_t, proc_t, diag_t, proc_t, med_t)
        bufs = ((rows0_v, sem0), (rows1_v, sem1),
                (rows2_v, sem2), (rows3_v, sem3))

        def idx_copy(look, slot):
            return pltpu.make_async_copy(codes[look].at[wid],
                                         idx_v.at[slot], sem_idx)

        def out_copy(look, slot):
            return pltpu.make_async_copy(acc_v.at[slot],
                                         out_hbm.at[look, pl.ds(base, bpw)],
                                         sem_out)

        idx_copy(0, 0).start()
        for look in range(NLOOK):
            slot = look % 2
            table = tables[look]
            idx_copy(look, slot).wait()
            if look + 1 < NLOOK:
                idx_copy(look + 1, 1 - slot).start()
            if look >= 2:
                out_copy(look - 2, slot).wait()

            def gather(g, buf, sem, table=table, slot=slot):
                return pltpu.make_async_copy(table.at[idx_v.at[slot, g]],
                                             buf, sem)

            def reduce_store(g, buf, slot=slot):
                def red(j, accs):
                    a0 = tuple(accs[d] + buf[j, pl.ds(d * 16, 16)]
                               for d in range(DCH))
                    a1 = tuple(accs[DCH + d] + buf[L + j, pl.ds(d * 16, 16)]
                               for d in range(DCH))
                    return a0 + a1

                zero = jnp.zeros((16,), jnp.float32)
                accs = lax.fori_loop(0, L, red, (zero,) * (2 * DCH),
                                     unroll=5)
                r0 = CHUNK_ROWS * g
                for d in range(DCH):
                    acc_v[slot, r0, pl.ds(d * 16, 16)] = accs[d]
                    acc_v[slot, r0 + 1, pl.ds(d * 16, 16)] = accs[DCH + d]

            for k in range(NBUF - 1):
                gather(k, *bufs[k]).start()

            def quad_body(qq, carry):
                g0 = NBUF * qq
                for k in range(NBUF):
                    nxt = g0 + k + NBUF - 1

                    @pl.when(nxt < nchunk)
                    def _(nxt=nxt, k=k):
                        gather(nxt, *bufs[(k + NBUF - 1) % NBUF]).start()

                    gather(g0 + k, *bufs[k]).wait()
                    reduce_store(g0 + k, bufs[k][0])
                return carry

            lax.fori_loop(0, nchunk // NBUF, quad_body, 0)
            out_copy(look, slot).start()

        out_copy(NLOOK - 2, (NLOOK - 2) % 2).wait()
        out_copy(NLOOK - 1, (NLOOK - 1) % 2).wait()

    return body


def _sc_pool(codes, diag_table, proc_table, med_table, nbatch):
    bpw = nbatch // NW
    nchunk = bpw // CHUNK_ROWS
    run = functools.partial(
        pl.kernel,
        mesh=plsc.VectorSubcoreMesh(core_axis_name="c", subcore_axis_name="s"),
        out_type=jax.ShapeDtypeStruct((NLOOK, nbatch, D), jnp.float32),
        scratch_types=[
            pltpu.VMEM((2, nchunk, IDX_MINOR), jnp.int32),
            pltpu.VMEM((IDX_MINOR, D), jnp.float32),
            pltpu.VMEM((IDX_MINOR, D), jnp.float32),
            pltpu.VMEM((IDX_MINOR, D), jnp.float32),
            pltpu.VMEM((IDX_MINOR, D), jnp.float32),
            pltpu.VMEM((2, bpw, D), jnp.float32),
            pltpu.SemaphoreType.DMA,
            pltpu.SemaphoreType.DMA,
            pltpu.SemaphoreType.DMA,
            pltpu.SemaphoreType.DMA,
            pltpu.SemaphoreType.DMA,
            pltpu.SemaphoreType.DMA,
        ],
    )(_make_sc_body(bpw, nchunk))
    return run(*codes, diag_table, proc_table, med_table)


OUT_RAW = 1000
OUT_PAD = 1024


MLP_BLK = 256


def _mlp_body(pooled, w1a, w1b, b1r, w2a, w2b, w2c, b2r, w3t, b3r, out):
    cur = pooled[0] @ w1a[:] + pooled[1] @ w1b[:] + b1r[:]
    prev = pooled[2] @ w1a[:] + pooled[3] @ w1b[:] + b1r[:]
    h = jnp.maximum(
        cur @ w2a[:] + prev @ w2b[:] + pooled[4] @ w2c[:] + b2r[:], 0.0)
    out[:] = jax.nn.sigmoid(h @ w3t[:] + b3r[:])


def _mlp_body_alias(unused_prev, *rest):
    _mlp_body(*rest)


def _weight_specs():
    full = lambda shape: pl.BlockSpec(shape, lambda i: (0,) * len(shape))
    return [full((D, D)), full((D, D)), full((1, D)),
            full((D, 256)), full((D, 256)), full((D, 256)), full((1, 256)),
            full((256, OUT_RAW)), full((1, OUT_RAW))]


def _mlp_part(pooled, weights, row_start, prev=None):
    nrows = pooled.shape[1]
    nblocks = nrows // MLP_BLK
    blk0 = row_start // MLP_BLK
    pooled_spec = pl.BlockSpec((NLOOK, MLP_BLK, D), lambda i: (0, i, 0))
    out_spec = pl.BlockSpec((MLP_BLK, OUT_RAW), lambda i, blk0=blk0: (blk0 + i, 0))
    in_specs = [pooled_spec] + _weight_specs()
    args = (pooled,) + weights
    body = _mlp_body
    aliases = {}
    if prev is not None:
        in_specs = [pl.BlockSpec(memory_space=pltpu.MemorySpace.HBM)] + in_specs
        args = (prev,) + args
        body = _mlp_body_alias
        aliases = {0: 0}
    return pl.pallas_call(
        body,
        grid=(nblocks,),
        in_specs=in_specs,
        out_specs=out_spec,
        out_shape=jax.ShapeDtypeStruct((B, OUT_RAW), jnp.float32),
        input_output_aliases=aliases,
    )(*args)


SPLIT = 768


def kernel(diag_codes, proc_codes, prev_diag_codes, prev_proc_codes,
           prev_med_codes, diag_table, proc_table, med_table,
           W1, b1, W2, b2, W3, b3):
    raw = (diag_codes, proc_codes, prev_diag_codes, prev_proc_codes,
           prev_med_codes)

    def codes_for(lo, n):
        nck = n // NW // CHUNK_ROWS
        return tuple(c.astype(jnp.int32)[lo:lo + n].reshape(
            NW, nck, IDX_MINOR) for c in raw)

    weights = (W1[:, :D].T, W1[:, D:].T, b1.reshape(1, D),
               W2[:, :D].T, W2[:, D:2 * D].T, W2[:, 2 * D:].T,
               b2.reshape(1, 256), W3.T, b3.reshape(1, OUT_RAW))

    pooled_a = _sc_pool(codes_for(0, SPLIT), diag_table, proc_table,
                        med_table, SPLIT)
    pooled_b = _sc_pool(codes_for(SPLIT, B - SPLIT), diag_table,
                        proc_table, med_table, B - SPLIT)
    out_a = _mlp_part(pooled_a, weights, 0)
    return _mlp_part(pooled_b, weights, SPLIT, prev=out_a)


# speedup vs baseline: 1.2669x; 1.2669x over previous
import functools

import jax
import jax.numpy as jnp
from jax import lax
from jax.experimental import pallas as pl
from jax.experimental.pallas import tpu as pltpu
from jax.experimental.pallas import tpu_sc as plsc

B = 1024
L = 50
D = 128
NLOOK = 5
NC, NS = 2, 16
NW = NC * NS
CHUNK_ROWS = 2
IDX_MINOR = 2 * L
DCH = D // 16


NBUF = 4


def _make_sc_body(bpw, nchunk):
    def body(c0, c1, c2, c3, c4, diag_t, proc_t, med_t, out_hbm,
             idx_v, rows0_v, rows1_v, rows2_v, rows3_v, acc_v, med_s,
             sem0, sem1, sem2, sem3, sem_idx, sem_out, sem_med):
        sid = lax.axis_index("s")
        wid = sid * NC + lax.axis_index("c")
        base = wid * bpw
        codes = (c0, c1, c2, c3, c4)
        tables = (diag_t, proc_t, diag_t, proc_t, med_s)

        med_stage = pltpu.make_async_copy(med_t, med_s, sem_med)

        @pl.when(sid == 0)
        def _():
            med_stage.start()
        bufs = ((rows0_v, sem0), (rows1_v, sem1),
                (rows2_v, sem2), (rows3_v, sem3))

        def idx_copy(look, slot):
            return pltpu.make_async_copy(codes[look].at[wid],
                                         idx_v.at[slot], sem_idx)

        def out_copy(look, slot):
            return pltpu.make_async_copy(acc_v.at[slot],
                                         out_hbm.at[look, pl.ds(base, bpw)],
                                         sem_out)

        idx_copy(0, 0).start()
        for look in range(NLOOK):
            slot = look % 2
            table = tables[look]
            if look == NLOOK - 1:
                @pl.when(sid == 0)
                def _():
                    med_stage.wait()
                plsc.subcore_barrier()
            idx_copy(look, slot).wait()
            if look + 1 < NLOOK:
                idx_copy(look + 1, 1 - slot).start()
            if look >= 2:
                out_copy(look - 2, slot).wait()

            def gather(g, buf, sem, table=table, slot=slot):
                return pltpu.make_async_copy(table.at[idx_v.at[slot, g]],
                                             buf, sem)

            def reduce_store(g, buf, slot=slot):
                def red(j, accs):
                    a0 = tuple(accs[d] + buf[j, pl.ds(d * 16, 16)]
                               for d in range(DCH))
                    a1 = tuple(accs[DCH + d] + buf[L + j, pl.ds(d * 16, 16)]
                               for d in range(DCH))
                    return a0 + a1

                zero = jnp.zeros((16,), jnp.float32)
                accs = lax.fori_loop(0, L, red, (zero,) * (2 * DCH),
                                     unroll=5)
                r0 = CHUNK_ROWS * g
                for d in range(DCH):
                    acc_v[slot, r0, pl.ds(d * 16, 16)] = accs[d]
                    acc_v[slot, r0 + 1, pl.ds(d * 16, 16)] = accs[DCH + d]

            for k in range(NBUF - 1):
                gather(k, *bufs[k]).start()

            def quad_body(qq, carry):
                g0 = NBUF * qq
                for k in range(NBUF):
                    nxt = g0 + k + NBUF - 1

                    @pl.when(nxt < nchunk)
                    def _(nxt=nxt, k=k):
                        gather(nxt, *bufs[(k + NBUF - 1) % NBUF]).start()

                    gather(g0 + k, *bufs[k]).wait()
                    reduce_store(g0 + k, bufs[k][0])
                return carry

            lax.fori_loop(0, nchunk // NBUF, quad_body, 0)
            out_copy(look, slot).start()

        out_copy(NLOOK - 2, (NLOOK - 2) % 2).wait()
        out_copy(NLOOK - 1, (NLOOK - 1) % 2).wait()

    return body


def _sc_pool(codes, diag_table, proc_table, med_table, nbatch):
    bpw = nbatch // NW
    nchunk = bpw // CHUNK_ROWS
    run = functools.partial(
        pl.kernel,
        mesh=plsc.VectorSubcoreMesh(core_axis_name="c", subcore_axis_name="s"),
        out_type=jax.ShapeDtypeStruct((NLOOK, nbatch, D), jnp.float32),
        scratch_types=[
            pltpu.VMEM((2, nchunk, IDX_MINOR), jnp.int32),
            pltpu.VMEM((IDX_MINOR, D), jnp.float32),
            pltpu.VMEM((IDX_MINOR, D), jnp.float32),
            pltpu.VMEM((IDX_MINOR, D), jnp.float32),
            pltpu.VMEM((IDX_MINOR, D), jnp.float32),
            pltpu.VMEM((2, bpw, D), jnp.float32),
            pltpu.VMEM_SHARED((1000, D), jnp.float32),
            pltpu.SemaphoreType.DMA,
            pltpu.SemaphoreType.DMA,
            pltpu.SemaphoreType.DMA,
            pltpu.SemaphoreType.DMA,
            pltpu.SemaphoreType.DMA,
            pltpu.SemaphoreType.DMA,
            pltpu.SemaphoreType.DMA,
        ],
    )(_make_sc_body(bpw, nchunk))
    return run(*codes, diag_table, proc_table, med_table)


OUT_RAW = 1000
OUT_PAD = 1024


def _mlp_body(pooled, w1a, w1b, b1r, w2a, w2b, w2c, b2r, w3t, b3r, out):
    cur = pooled[0] @ w1a[:] + pooled[1] @ w1b[:] + b1r[:]
    prev = pooled[2] @ w1a[:] + pooled[3] @ w1b[:] + b1r[:]
    h = jnp.maximum(
        cur @ w2a[:] + prev @ w2b[:] + pooled[4] @ w2c[:] + b2r[:], 0.0)
    out[:] = jax.nn.sigmoid(h @ w3t[:] + b3r[:])


def _mlp(pooled, w1a, w1b, b1r, w2a, w2b, w2c, b2r, w3t, b3r):
    nrows = pooled.shape[1]
    return pl.pallas_call(
        _mlp_body,
        out_shape=jax.ShapeDtypeStruct((nrows, OUT_RAW), jnp.float32),
    )(pooled, w1a, w1b, b1r, w2a, w2b, w2c, b2r, w3t, b3r)


def kernel(diag_codes, proc_codes, prev_diag_codes, prev_proc_codes,
           prev_med_codes, diag_table, proc_table, med_table,
           W1, b1, W2, b2, W3, b3):
    nchunk = B // NW // CHUNK_ROWS
    codes = tuple(
        c.astype(jnp.int32).reshape(NW, nchunk, IDX_MINOR)
        for c in (diag_codes, proc_codes, prev_diag_codes,
                  prev_proc_codes, prev_med_codes))

    w1a = W1[:, :D].T
    w1b = W1[:, D:].T
    b1r = b1.reshape(1, D)
    w2a = W2[:, :D].T
    w2b = W2[:, D:2 * D].T
    w2c = W2[:, 2 * D:].T
    b2r = b2.reshape(1, 256)
    w3t = W3.T
    b3r = b3.reshape(1, OUT_RAW)

    pooled = _sc_pool(codes, diag_table, proc_table, med_table, B)
    return _mlp(pooled, w1a, w1b, b1r, w2a, w2b, w2c, b2r, w3t, b3r)


# speedup vs baseline: 1.2675x; 1.0005x over previous
import functools

import jax
import jax.numpy as jnp
from jax import lax
from jax.experimental import pallas as pl
from jax.experimental.pallas import tpu as pltpu
from jax.experimental.pallas import tpu_sc as plsc

B = 1024
L = 50
D = 128
NLOOK = 5
NC, NS = 2, 16
NW = NC * NS
CHUNK_ROWS = 2
IDX_MINOR = 2 * L
DCH = D // 16


NBUF = 4


def _make_sc_body(bpw, nchunk):
    def body(c0, c1, c2, c3, c4, diag_t, proc_t, med_t, out_hbm,
             idx_v, rows0_v, rows1_v, rows2_v, rows3_v, acc_v, med_s,
             sem0, sem1, sem2, sem3, sem_idx, sem_out, sem_med):
        sid = lax.axis_index("s")
        wid = sid * NC + lax.axis_index("c")
        base = wid * bpw
        codes = (c0, c1, c2, c3, c4)
        tables = (diag_t, proc_t, diag_t, proc_t, med_s)

        med_stage = pltpu.make_async_copy(med_t, med_s, sem_med)

        @pl.when(sid == 0)
        def _():
            med_stage.start()
        bufs = ((rows0_v, sem0), (rows1_v, sem1),
                (rows2_v, sem2), (rows3_v, sem3))

        def idx_copy(look, slot):
            return pltpu.make_async_copy(codes[look].at[wid],
                                         idx_v.at[slot], sem_idx)

        def out_copy(look, slot):
            return pltpu.make_async_copy(acc_v.at[slot],
                                         out_hbm.at[look, pl.ds(base, bpw)],
                                         sem_out)

        idx_copy(0, 0).start()
        for look in range(NLOOK):
            slot = look % 2
            table = tables[look]
            if look == NLOOK - 1:
                @pl.when(sid == 0)
                def _():
                    med_stage.wait()
                plsc.subcore_barrier()
            idx_copy(look, slot).wait()
            if look + 1 < NLOOK:
                idx_copy(look + 1, 1 - slot).start()
            if look >= 2:
                out_copy(look - 2, slot).wait()

            def gather(g, buf, sem, table=table, slot=slot):
                return pltpu.make_async_copy(table.at[idx_v.at[slot, g]],
                                             buf, sem)

            def reduce_store(g, buf, slot=slot):
                def red(j, accs):
                    a0 = tuple(accs[d] + buf[j, pl.ds(d * 16, 16)]
                               for d in range(DCH))
                    a1 = tuple(accs[DCH + d] + buf[L + j, pl.ds(d * 16, 16)]
                               for d in range(DCH))
                    return a0 + a1

                zero = jnp.zeros((16,), jnp.float32)
                accs = lax.fori_loop(0, L, red, (zero,) * (2 * DCH),
                                     unroll=5)
                r0 = CHUNK_ROWS * g
                for d in range(DCH):
                    acc_v[slot, r0, pl.ds(d * 16, 16)] = accs[d]
                    acc_v[slot, r0 + 1, pl.ds(d * 16, 16)] = accs[DCH + d]

            for k in range(NBUF - 1):
                gather(k, *bufs[k]).start()

            def quad_body(qq, carry):
                g0 = NBUF * qq
                for k in range(NBUF):
                    nxt = g0 + k + NBUF - 1

                    @pl.when(nxt < nchunk)
                    def _(nxt=nxt, k=k):
                        gather(nxt, *bufs[(k + NBUF - 1) % NBUF]).start()

                    gather(g0 + k, *bufs[k]).wait()
                    reduce_store(g0 + k, bufs[k][0])
                return carry

            lax.fori_loop(0, nchunk // NBUF, quad_body, 0)
            out_copy(look, slot).start()

        out_copy(NLOOK - 2, (NLOOK - 2) % 2).wait()
        out_copy(NLOOK - 1, (NLOOK - 1) % 2).wait()

    return body


def _sc_pool(codes, diag_table, proc_table, med_table, nbatch):
    bpw = nbatch // NW
    nchunk = bpw // CHUNK_ROWS
    run = functools.partial(
        pl.kernel,
        mesh=plsc.VectorSubcoreMesh(core_axis_name="c", subcore_axis_name="s"),
        out_type=jax.ShapeDtypeStruct((NLOOK, nbatch, D), jnp.float32),
        scratch_types=[
            pltpu.VMEM((2, nchunk, IDX_MINOR), jnp.int32),
            pltpu.VMEM((IDX_MINOR, D), jnp.float32),
            pltpu.VMEM((IDX_MINOR, D), jnp.float32),
            pltpu.VMEM((IDX_MINOR, D), jnp.float32),
            pltpu.VMEM((IDX_MINOR, D), jnp.float32),
            pltpu.VMEM((2, bpw, D), jnp.float32),
            pltpu.VMEM_SHARED((1000, D), jnp.float32),
            pltpu.SemaphoreType.DMA,
            pltpu.SemaphoreType.DMA,
            pltpu.SemaphoreType.DMA,
            pltpu.SemaphoreType.DMA,
            pltpu.SemaphoreType.DMA,
            pltpu.SemaphoreType.DMA,
            pltpu.SemaphoreType.DMA,
        ],
    )(_make_sc_body(bpw, nchunk))
    return run(*codes, diag_table, proc_table, med_table)


OUT_RAW = 1000


def _mlp_body(pooled, w1a, w1b, b1r, w2a, w2b, w2c, b2r, w3t, b3r, out):
    cur = pooled[0] @ w1a[:] + pooled[1] @ w1b[:] + b1r[:]
    prev = pooled[2] @ w1a[:] + pooled[3] @ w1b[:] + b1r[:]
    h = jnp.maximum(
        cur @ w2a[:] + prev @ w2b[:] + pooled[4] @ w2c[:] + b2r[:], 0.0)
    out[:] = jax.nn.sigmoid(h @ w3t[:] + b3r[:])


def _mlp(pooled, w1a, w1b, b1r, w2a, w2b, w2c, b2r, w3t, b3r):
    nrows = pooled.shape[1]
    return pl.pallas_call(
        _mlp_body,
        out_shape=jax.ShapeDtypeStruct((nrows, OUT_RAW), jnp.float32),
    )(pooled, w1a, w1b, b1r, w2a, w2b, w2c, b2r, w3t, b3r)


def kernel(diag_codes, proc_codes, prev_diag_codes, prev_proc_codes,
           prev_med_codes, diag_table, proc_table, med_table,
           W1, b1, W2, b2, W3, b3):
    nchunk = B // NW // CHUNK_ROWS
    codes = tuple(
        c.astype(jnp.int32).reshape(NW, nchunk, IDX_MINOR)
        for c in (diag_codes, proc_codes, prev_diag_codes,
                  prev_proc_codes, prev_med_codes))

    w1a = W1[:, :D].T
    w1b = W1[:, D:].T
    b1r = b1.reshape(1, D)
    w2a = W2[:, :D].T
    w2b = W2[:, D:2 * D].T
    w2c = W2[:, 2 * D:].T
    b2r = b2.reshape(1, 256)
    w3t = W3.T
    b3r = b3.reshape(1, OUT_RAW)

    pooled = _sc_pool(codes, diag_table, proc_table, med_table, B)
    return _mlp(pooled, w1a, w1b, b1r, w2a, w2b, w2c, b2r, w3t, b3r)
